# Initial kernel scaffold; baseline (speedup 1.0000x reference)
import jax
import jax.numpy as jnp
from jax.experimental import pallas as pl


def kernel(feat_s, y_s, feat_q):
    raise NotImplementedError("write your pallas kernel here")


# speedup vs baseline: 2.3294x; 2.3294x over previous
import math
import functools

import jax
import jax.numpy as jnp
import numpy as np
from jax.experimental import pallas as pl
from jax.experimental.pallas import tpu as pltpu

NQ = 300
L = 5
DIM = 512
NP_TRI = NQ * (NQ - 1) // 2
VMAX = NP_TRI + L
NPAD = NQ + VMAX
NP_PAD = 46080
RT = 128
CT = 1024
GI = NP_PAD // RT
GJ = NP_PAD // CT
KNN = max(1, int(math.log2(NPAD)))
KSEL = KNN + 1


def _cd(a, b):
    d2 = jnp.sum(a * a, 1)[:, None] + jnp.sum(b * b, 1)[None, :] - 2.0 * (a @ b.T)
    return jnp.sqrt(jnp.clip(d2, 1e-12))


def _knn_body(fr_ref, fc_ref, pen_ref, nnd_ref, nni_ref, kbuf):
    j = pl.program_id(1)
    fr = fr_ref[...]
    fc = fc_ref[...]
    mm = jax.lax.dot_general(fr, fc, (((1,), (1,)), ((), ())),
                             preferred_element_type=jnp.float32)
    rn = jnp.sum(fr * fr, axis=1, keepdims=True)
    cn = jnp.sum(fc * fc, axis=1, keepdims=True)
    d2 = rn + jnp.transpose(cn) - 2.0 * mm
    kbuf[:, pl.ds(j * CT, CT)] = jnp.maximum(d2, 1e-12) + pen_ref[...]

    @pl.when(j == GJ - 1)
    def _select():
        colid = jax.lax.broadcasted_iota(jnp.int32, (RT, NP_PAD), 1)
        for s in range(KSEL):
            k = kbuf[...]
            m = jnp.min(k, axis=1)
            hit = k == m[:, None]
            idx = jnp.min(jnp.where(hit, colid, NP_PAD), axis=1)
            nnd_ref[:, s:s + 1] = jnp.sqrt(m)[:, None]
            nni_ref[:, s:s + 1] = idx[:, None]
            kbuf[...] = jnp.where(colid == idx[:, None], jnp.inf, k)


@functools.partial(jax.jit, static_argnums=())
def _knn(feat_pad, pen):
    nnd, nni = pl.pallas_call(
        _knn_body,
        grid=(GI, GJ),
        in_specs=[
            pl.BlockSpec((RT, DIM), lambda i, j: (i, 0)),
            pl.BlockSpec((CT, DIM), lambda i, j: (j, 0)),
            pl.BlockSpec((1, CT), lambda i, j: (0, j)),
        ],
        out_specs=[
            pl.BlockSpec((RT, 128), lambda i, j: (i, 0)),
            pl.BlockSpec((RT, 128), lambda i, j: (i, 0)),
        ],
        out_shape=[
            jax.ShapeDtypeStruct((NP_PAD, 128), jnp.float32),
            jax.ShapeDtypeStruct((NP_PAD, 128), jnp.int32),
        ],
        scratch_shapes=[pltpu.VMEM((RT, NP_PAD), jnp.float32)],
    )(feat_pad, feat_pad, pen)
    return nnd[:NPAD, :KSEL], nni[:NPAD, :KSEL]


def kernel(feat_s, y_s, feat_q):
    labels = jnp.arange(L)
    onehot_s = y_s[:, None] == labels[None, :]
    cnt_s = onehot_s.sum(0)
    present = cnt_s > 0
    prototypes = (onehot_s.astype(feat_s.dtype).T @ feat_s) / jnp.maximum(
        cnt_s, 1).astype(feat_s.dtype)[:, None]
    dist_q = _cd(feat_q, prototypes)
    y_boot = jnp.argmin(jnp.where(present[None, :], dist_q, jnp.inf), 1)
    qn = feat_q / jnp.maximum(jnp.linalg.norm(feat_q, axis=1, keepdims=True), 1e-12)
    cos = qn @ qn.T
    bootmask = y_boot[None, :] == labels[:, None]
    m_boot = bootmask.sum(1)
    pairmask2 = bootmask[:, :, None] & bootmask[:, None, :]
    vals = jnp.sort(jnp.where(pairmask2, cos[None], jnp.inf).reshape(L, NQ * NQ), axis=1)
    midx = jnp.clip((m_boot * m_boot - 1) // 2, 0, NQ * NQ - 1)
    thresh = jnp.take_along_axis(vals, midx[:, None], axis=1)[:, 0]
    ii0_np, jj0_np = np.triu_indices(NQ, 1)
    ii0 = jnp.asarray(ii0_np)
    jj0 = jnp.asarray(jj0_np)
    pairvalid = (bootmask[:, ii0_np] & bootmask[:, jj0_np]
                 & (cos[ii0, jj0][None, :] >= thresh[:, None]))
    dq_by_c = jnp.where(bootmask, dist_q.T, jnp.inf)
    kb = jnp.argmin(dq_by_c, axis=1)
    blendfeat = 0.5 * prototypes + 0.5 * feat_q[kb]
    blendvalid = m_boot >= 2
    valid = jnp.concatenate([pairvalid, blendvalid[:, None]], axis=1).reshape(
        L * (NP_TRI + 1))
    order = jnp.argsort(jnp.where(valid, 0, 1))
    sel = order[:VMAX]
    slot_c = sel // (NP_TRI + 1)
    slot_r = sel % (NP_TRI + 1)
    isb = slot_r == NP_TRI
    pr = jnp.minimum(slot_r, NP_TRI - 1)
    pairfeat = 0.5 * feat_q[ii0[pr]] + 0.5 * feat_q[jj0[pr]]
    virt = jnp.where(isb[:, None], blendfeat[slot_c], pairfeat)
    virt = jnp.where(valid[sel][:, None], virt, 0.0)
    feat_all = jnp.concatenate([feat_q, virt], 0)
    n_real = NQ + valid.sum()
    real = jnp.arange(NPAD) < n_real
    kk = jnp.floor(jnp.log2(n_real.astype(jnp.float32))).astype(n_real.dtype)
    kk = jnp.where(2 ** (kk + 1) <= n_real, kk + 1, kk)
    kk = jnp.where(2 ** kk > n_real, kk - 1, kk)
    kk = jnp.maximum(1, kk)

    feat_pad = jnp.pad(feat_all, ((0, NP_PAD - NPAD), (0, 0)))
    pen = jnp.where(jnp.arange(NP_PAD) < n_real, 0.0, jnp.inf)[None, :]
    nn_d, nn_i = _knn(feat_pad, pen)

    sigma = jnp.take_along_axis(nn_d, jnp.full((NPAD, 1), 0) + kk, axis=1)[:, 0] + 1e-8
    nbrs = nn_i[:, 1:]
    dnb = nn_d[:, 1:]
    ranks = jnp.arange(1, KNN + 1)
    active = (ranks[None, :] <= kk) & real[:, None]
    expnt = jnp.where(active, -dnb / (sigma[:, None] * sigma[nbrs]), -jnp.inf)
    w = jnp.exp(expnt)

    rows = jnp.broadcast_to(jnp.arange(NPAD)[:, None], (NPAD, KNN)).reshape(-1)
    cols = nbrs.reshape(-1)
    wf = 0.5 * w.reshape(-1)
    dst = jnp.concatenate([rows, cols])
    src = jnp.concatenate([cols, rows])
    we = jnp.concatenate([wf, wf])
    perm = jnp.argsort(dst)
    dsts = dst[perm]
    srcs = src[perm]
    wes = we[perm]
    wsum = jax.ops.segment_sum(wes, dsts, num_segments=NPAD,
                               indices_are_sorted=True)
    d_inv = 1.0 / (wsum + 1e-8)

    dp = _cd(feat_all, prototypes)
    d = jnp.min(jnp.where(present[None, :], dp, jnp.inf), 1)
    ds = jnp.sort(jnp.where(real, d, jnp.inf))
    med = ds[(n_real - 1) // 2]
    lam = jnp.exp(-d ** 2 / (2.0 * med ** 2 + 1e-8))
    a = jnp.where(present[None, :], dp ** 2, jnp.inf)
    y0 = jax.nn.softmax(-a, axis=1)
    coef = (lam * d_inv)[:, None]

    def body(t, carry):
        y, done = carry
        wy = jax.ops.segment_sum(wes[:, None] * y[srcs], dsts,
                                 num_segments=NPAD, indices_are_sorted=True)
        y_new = jax.nn.softmax(-a + coef * wy, axis=1)
        diff = jnp.max(jnp.where(real[:, None], jnp.abs(y_new - y), 0.0))
        stop = diff < 1e-4
        y = jnp.where(jnp.logical_or(done, stop), y, y_new)
        return y, jnp.logical_or(done, stop)

    y, _ = jax.lax.fori_loop(0, 50, body, (y0, jnp.array(False)))
    lbl = jnp.argmax(y[:NQ], axis=1)
    mapping = jnp.cumsum(present.astype(lbl.dtype)) - 1
    return mapping[lbl]
